# baseline (speedup 1.0000x reference)
import functools

import jax
import jax.numpy as jnp
from jax import lax
from jax.experimental import pallas as pl
from jax.experimental.pallas import tpu as pltpu
from jax.experimental.pallas import tpu_sc as plsc

_VOCAB = 1000000
_EMBED = 64
_NCLS = 10
_B = 4096
_L = 200
_C0 = 104
_C1 = _L - _C0
_NC = 2
_NS = 16
_NW = _NC * _NS
_SEQ_PER_W = _B // _NW
_LANES = 16


_NBUF = 4


def _sc_body(table_hbm, idx_hbm, out_hbm, idx_v, rows_v, outbuf, sems):
    wid = lax.axis_index("s") * _NC + lax.axis_index("c")
    base = wid * _SEQ_PER_W
    pltpu.sync_copy(idx_hbm.at[pl.ds(base, _SEQ_PER_W)], idx_v)

    nvec = _EMBED // _LANES

    def start(slot, s):
        pltpu.async_copy(table_hbm.at[idx_v.at[s]], rows_v.at[slot],
                         sems.at[slot])

    def wait(slot, s):
        pltpu.make_async_copy(table_hbm.at[idx_v.at[s]], rows_v.at[slot],
                              sems.at[slot]).wait()

    def reduce_slot(slot, accs):
        def row_body(r, accs):
            return tuple(
                a + rows_v[slot, r, pl.ds(e * _LANES, _LANES)]
                for e, a in enumerate(accs)
            )
        return lax.fori_loop(0, _L, row_body, accs, unroll=8)

    for j in range(_NBUF):
        start(j, j)

    ngrp = _SEQ_PER_W // _NBUF

    def grp_body(g, carry):
        for j in range(_NBUF):
            s = _NBUF * g + j
            wait(j, s)

            @pl.when(g < ngrp - 1)
            def _():
                start(j, s + _NBUF)

            accs = tuple(jnp.zeros((_LANES,), jnp.float32)
                         for _ in range(nvec))
            accs = reduce_slot(j, accs)
            for e in range(nvec):
                outbuf[s, pl.ds(e * _LANES, _LANES)] = accs[e]
        return carry

    lax.fori_loop(0, ngrp, grp_body, 0)
    pltpu.sync_copy(outbuf, out_hbm.at[pl.ds(base, _SEQ_PER_W)])


_sc_gather_sum = functools.partial(
    pl.kernel,
    out_type=jax.ShapeDtypeStruct((_B, _EMBED), jnp.float32),
    mesh=plsc.VectorSubcoreMesh(
        core_axis_name="c", subcore_axis_name="s",
        num_cores=_NC, num_subcores=_NS),
    scratch_types=[
        pltpu.VMEM((_SEQ_PER_W, _L), jnp.int32),
        pltpu.VMEM((_NBUF, _L, _EMBED), jnp.float32),
        pltpu.VMEM((_SEQ_PER_W, _EMBED), jnp.float32),
        pltpu.SemaphoreType.DMA((_NBUF,)),
    ],
    compiler_params=pltpu.CompilerParams(use_tc_tiling_on_sc=False),
)(_sc_body)


_TBLK = 1024
_KSPLIT = 490 * _TBLK


def _tr_body(x0_ref, x1_ref, out_ref):
    eye = jnp.eye(_EMBED, dtype=jnp.float32)
    dn = (((0,), (0,)), ((), ()))
    out_ref[:, 0:_EMBED] = lax.dot_general(
        x0_ref[...], eye, dn, preferred_element_type=jnp.float32)
    out_ref[:, _EMBED:2 * _EMBED] = lax.dot_general(
        x1_ref[...], eye, dn, preferred_element_type=jnp.float32)


def _tc_transpose(emb_t):
    grid = _KSPLIT // _TBLK
    return pl.pallas_call(
        _tr_body,
        grid=(grid,),
        in_specs=[
            pl.BlockSpec((_EMBED, _TBLK), lambda i: (0, i)),
            pl.BlockSpec(
                (_EMBED, _TBLK),
                lambda i: (0, jnp.minimum(grid + i, (_VOCAB - 1) // _TBLK)),
            ),
        ],
        out_specs=pl.BlockSpec((_TBLK, 2 * _EMBED), lambda i: (i, 0)),
        out_shape=jax.ShapeDtypeStruct((_KSPLIT, 2 * _EMBED), jnp.float32),
    )(emb_t, emb_t)


def _tc_body(summed_ref, seq_ref, wt_ref, b_ref, out_ref):
    cnt = jnp.sum((seq_ref[...] != 0).astype(jnp.float32), axis=1,
                  keepdims=True)
    cnt = jnp.maximum(cnt, 1.0)
    acc = jnp.dot(summed_ref[...], wt_ref[...],
                  preferred_element_type=jnp.float32)
    out_ref[...] = acc / cnt + b_ref[...]


def _tc_classifier(summed, seqs, wt, b2):
    blk = 1024
    grid = _B // blk
    return pl.pallas_call(
        _tc_body,
        grid=(grid,),
        in_specs=[
            pl.BlockSpec((blk, _EMBED), lambda i: (i, 0)),
            pl.BlockSpec((blk, _L), lambda i: (i, 0)),
            pl.BlockSpec((_EMBED, _NCLS), lambda i: (0, 0)),
            pl.BlockSpec((1, _NCLS), lambda i: (0, 0)),
        ],
        out_specs=pl.BlockSpec((blk, _NCLS), lambda i: (i, 0)),
        out_shape=jax.ShapeDtypeStruct((_B, _NCLS), jnp.float32),
    )(summed, seqs, wt, b2)


def kernel(sequences, emb_table, W, b):
    seqs = sequences.astype(jnp.int32)
    wide = _tc_transpose(emb_table.T)
    t_lin = wide.reshape(2 * _KSPLIT, _EMBED)
    hi = (seqs >= _KSPLIT).astype(jnp.int32)
    gidx = 2 * (seqs - _KSPLIT * hi) + hi
    summed = _sc_gather_sum(t_lin, gidx)
    return _tc_classifier(summed, seqs, W.T, b.reshape(1, _NCLS))


# speedup vs baseline: 1.5539x; 1.5539x over previous
import functools

import jax
import jax.numpy as jnp
from jax import lax
from jax.experimental import pallas as pl
from jax.experimental.pallas import tpu as pltpu
from jax.experimental.pallas import tpu_sc as plsc

_VOCAB = 1000000
_EMBED = 64
_NCLS = 10
_B = 4096
_L = 200
_C0 = 104
_C1 = _L - _C0
_NC = 2
_NS = 16
_NW = _NC * _NS
_SEQ_PER_W = _B // _NW
_LANES = 16


_NBUF = 4


def _sc_body(table_hbm, idx_hbm, out_hbm, idx_v, rows_v, outbuf, sems):
    wid = lax.axis_index("s") * _NC + lax.axis_index("c")
    base = wid * _SEQ_PER_W
    pltpu.sync_copy(idx_hbm.at[pl.ds(base, _SEQ_PER_W)], idx_v)

    nvec = _EMBED // _LANES

    def start(slot, s):
        pltpu.async_copy(table_hbm.at[idx_v.at[s]], rows_v.at[slot],
                         sems.at[slot])

    def wait(slot, s):
        pltpu.make_async_copy(table_hbm.at[idx_v.at[s]], rows_v.at[slot],
                              sems.at[slot]).wait()

    def reduce_slot(slot, accs):
        def row_body(r, accs):
            return tuple(
                a + rows_v[slot, r, pl.ds(e * _LANES, _LANES)]
                for e, a in enumerate(accs)
            )
        return lax.fori_loop(0, _L, row_body, accs, unroll=8)

    for j in range(_NBUF):
        start(j, j)

    ngrp = _SEQ_PER_W // _NBUF

    def grp_body(g, carry):
        for j in range(_NBUF):
            s = _NBUF * g + j
            wait(j, s)

            @pl.when(g < ngrp - 1)
            def _():
                start(j, s + _NBUF)

            accs = tuple(jnp.zeros((_LANES,), jnp.float32)
                         for _ in range(nvec))
            accs = reduce_slot(j, accs)
            for e in range(nvec):
                outbuf[s, pl.ds(e * _LANES, _LANES)] = accs[e]
        return carry

    lax.fori_loop(0, ngrp, grp_body, 0)
    pltpu.sync_copy(outbuf, out_hbm.at[pl.ds(base, _SEQ_PER_W)])


_sc_gather_sum = functools.partial(
    pl.kernel,
    out_type=jax.ShapeDtypeStruct((_B, _EMBED), jnp.float32),
    mesh=plsc.VectorSubcoreMesh(
        core_axis_name="c", subcore_axis_name="s",
        num_cores=_NC, num_subcores=_NS),
    scratch_types=[
        pltpu.VMEM((_SEQ_PER_W, _L), jnp.int32),
        pltpu.VMEM((_NBUF, _L, _EMBED), jnp.float32),
        pltpu.VMEM((_SEQ_PER_W, _EMBED), jnp.float32),
        pltpu.SemaphoreType.DMA((_NBUF,)),
    ],
    compiler_params=pltpu.CompilerParams(use_tc_tiling_on_sc=False),
)(_sc_body)


_TBLK = 4096
_KSPLIT = 123 * _TBLK


def _tr_body(x0_ref, x1_ref, out_ref):
    out_ref[:, 0:_EMBED] = x0_ref[...].T
    out_ref[:, _EMBED:2 * _EMBED] = x1_ref[...].T


def _tc_transpose(emb_t):
    grid = _KSPLIT // _TBLK
    return pl.pallas_call(
        _tr_body,
        grid=(grid,),
        in_specs=[
            pl.BlockSpec((_EMBED, _TBLK), lambda i: (0, i)),
            pl.BlockSpec(
                (_EMBED, _TBLK),
                lambda i: (0, jnp.minimum(grid + i, (_VOCAB - 1) // _TBLK)),
            ),
        ],
        out_specs=pl.BlockSpec((_TBLK, 2 * _EMBED), lambda i: (i, 0)),
        out_shape=jax.ShapeDtypeStruct((_KSPLIT, 2 * _EMBED), jnp.float32),
    )(emb_t, emb_t)


def _tc_body(summed_ref, seq_ref, wt_ref, b_ref, out_ref):
    cnt = jnp.sum((seq_ref[...] != 0).astype(jnp.float32), axis=1,
                  keepdims=True)
    cnt = jnp.maximum(cnt, 1.0)
    acc = jnp.dot(summed_ref[...], wt_ref[...],
                  preferred_element_type=jnp.float32)
    out_ref[...] = acc / cnt + b_ref[...]


def _tc_classifier(summed, seqs, wt, b2):
    blk = 1024
    grid = _B // blk
    return pl.pallas_call(
        _tc_body,
        grid=(grid,),
        in_specs=[
            pl.BlockSpec((blk, _EMBED), lambda i: (i, 0)),
            pl.BlockSpec((blk, _L), lambda i: (i, 0)),
            pl.BlockSpec((_EMBED, _NCLS), lambda i: (0, 0)),
            pl.BlockSpec((1, _NCLS), lambda i: (0, 0)),
        ],
        out_specs=pl.BlockSpec((blk, _NCLS), lambda i: (i, 0)),
        out_shape=jax.ShapeDtypeStruct((_B, _NCLS), jnp.float32),
    )(summed, seqs, wt, b2)


def kernel(sequences, emb_table, W, b):
    seqs = sequences.astype(jnp.int32)
    wide = _tc_transpose(emb_table.T)
    t_lin = wide.reshape(2 * _KSPLIT, _EMBED)
    hi = (seqs >= _KSPLIT).astype(jnp.int32)
    gidx = 2 * (seqs - _KSPLIT * hi) + hi
    summed = _sc_gather_sum(t_lin, gidx)
    return _tc_classifier(summed, seqs, W.T, b.reshape(1, _NCLS))


# speedup vs baseline: 1.6942x; 1.0903x over previous
import functools

import jax
import jax.numpy as jnp
from jax import lax
from jax.experimental import pallas as pl
from jax.experimental.pallas import tpu as pltpu
from jax.experimental.pallas import tpu_sc as plsc

_VOCAB = 1000000
_EMBED = 64
_NCLS = 10
_B = 4096
_L = 200
_C0 = 104
_C1 = _L - _C0
_NC = 2
_NS = 16
_NW = _NC * _NS
_SEQ_PER_W = _B // _NW
_LANES = 16


_NBUF = 4


def _sc_body(table_hbm, idx_hbm, out_hbm, idx_v, rows_v, outbuf, sems):
    wid = lax.axis_index("s") * _NC + lax.axis_index("c")
    base = wid * _SEQ_PER_W
    pltpu.sync_copy(idx_hbm.at[pl.ds(base, _SEQ_PER_W)], idx_v)

    nvec = _EMBED // _LANES

    def start(slot, s):
        pltpu.async_copy(table_hbm.at[idx_v.at[s]], rows_v.at[slot],
                         sems.at[slot])

    def wait(slot, s):
        pltpu.make_async_copy(table_hbm.at[idx_v.at[s]], rows_v.at[slot],
                              sems.at[slot]).wait()

    def reduce_slot(slot, accs):
        def row_body(r, accs):
            return tuple(
                a + rows_v[slot, r, pl.ds(e * _LANES, _LANES)]
                for e, a in enumerate(accs)
            )
        return lax.fori_loop(0, _L, row_body, accs, unroll=8)

    for j in range(_NBUF):
        start(j, j)

    ngrp = _SEQ_PER_W // _NBUF

    def grp_body(g, carry):
        for j in range(_NBUF):
            s = _NBUF * g + j
            wait(j, s)

            @pl.when(g < ngrp - 1)
            def _():
                start(j, s + _NBUF)

            accs = tuple(jnp.zeros((_LANES,), jnp.float32)
                         for _ in range(nvec))
            accs = reduce_slot(j, accs)
            for e in range(nvec):
                outbuf[s, pl.ds(e * _LANES, _LANES)] = accs[e]
        return carry

    lax.fori_loop(0, ngrp, grp_body, 0)
    pltpu.sync_copy(outbuf, out_hbm.at[pl.ds(base, _SEQ_PER_W)])


_sc_gather_sum = functools.partial(
    pl.kernel,
    out_type=jax.ShapeDtypeStruct((_B, _EMBED), jnp.float32),
    mesh=plsc.VectorSubcoreMesh(
        core_axis_name="c", subcore_axis_name="s",
        num_cores=_NC, num_subcores=_NS),
    scratch_types=[
        pltpu.VMEM((_SEQ_PER_W, _L), jnp.int32),
        pltpu.VMEM((_NBUF, _L, _EMBED), jnp.float32),
        pltpu.VMEM((_SEQ_PER_W, _EMBED), jnp.float32),
        pltpu.SemaphoreType.DMA((_NBUF,)),
    ],
    compiler_params=pltpu.CompilerParams(use_tc_tiling_on_sc=False),
)(_sc_body)


_TBLK = 8192
_KSPLIT = 62 * _TBLK


def _tr_body(x0_ref, x1_ref, out_ref):
    out_ref[:, 0:_EMBED] = x0_ref[...].T
    out_ref[:, _EMBED:2 * _EMBED] = x1_ref[...].T


def _tc_transpose(emb_t):
    grid = _KSPLIT // _TBLK
    return pl.pallas_call(
        _tr_body,
        grid=(grid,),
        in_specs=[
            pl.BlockSpec((_EMBED, _TBLK), lambda i: (0, i)),
            pl.BlockSpec(
                (_EMBED, _TBLK),
                lambda i: (0, jnp.minimum(grid + i, (_VOCAB - 1) // _TBLK)),
            ),
        ],
        out_specs=pl.BlockSpec((_TBLK, 2 * _EMBED), lambda i: (i, 0)),
        out_shape=jax.ShapeDtypeStruct((_KSPLIT, 2 * _EMBED), jnp.float32),
    )(emb_t, emb_t)


def _tc_body(summed_ref, seq_ref, wt_ref, b_ref, out_ref):
    cnt = jnp.sum((seq_ref[...] != 0).astype(jnp.float32), axis=1,
                  keepdims=True)
    cnt = jnp.maximum(cnt, 1.0)
    acc = jnp.dot(summed_ref[...], wt_ref[...],
                  preferred_element_type=jnp.float32)
    out_ref[...] = acc / cnt + b_ref[...]


def _tc_classifier(summed, seqs, wt, b2):
    blk = 1024
    grid = _B // blk
    return pl.pallas_call(
        _tc_body,
        grid=(grid,),
        in_specs=[
            pl.BlockSpec((blk, _EMBED), lambda i: (i, 0)),
            pl.BlockSpec((blk, _L), lambda i: (i, 0)),
            pl.BlockSpec((_EMBED, _NCLS), lambda i: (0, 0)),
            pl.BlockSpec((1, _NCLS), lambda i: (0, 0)),
        ],
        out_specs=pl.BlockSpec((blk, _NCLS), lambda i: (i, 0)),
        out_shape=jax.ShapeDtypeStruct((_B, _NCLS), jnp.float32),
    )(summed, seqs, wt, b2)


def kernel(sequences, emb_table, W, b):
    seqs = sequences.astype(jnp.int32)
    wide = _tc_transpose(emb_table.T)
    t_lin = wide.reshape(2 * _KSPLIT, _EMBED)
    hi = (seqs >= _KSPLIT).astype(jnp.int32)
    gidx = 2 * (seqs - _KSPLIT * hi) + hi
    summed = _sc_gather_sum(t_lin, gidx)
    return _tc_classifier(summed, seqs, W.T, b.reshape(1, _NCLS))


# speedup vs baseline: 1.7713x; 1.0455x over previous
import functools

import jax
import jax.numpy as jnp
from jax import lax
from jax.experimental import pallas as pl
from jax.experimental.pallas import tpu as pltpu
from jax.experimental.pallas import tpu_sc as plsc

_VOCAB = 1000000
_EMBED = 64
_NCLS = 10
_B = 4096
_L = 200
_C0 = 104
_C1 = _L - _C0
_NC = 2
_NS = 16
_NW = _NC * _NS
_SEQ_PER_W = _B // _NW
_LANES = 16


_NBUF = 4


def _sc_body(table_hbm, idx_hbm, out_hbm, idx_v, rows_v, outbuf, sems):
    wid = lax.axis_index("s") * _NC + lax.axis_index("c")
    base = wid * _SEQ_PER_W
    pltpu.sync_copy(idx_hbm.at[pl.ds(base, _SEQ_PER_W)], idx_v)

    nvec = _EMBED // _LANES

    def start(slot, s):
        pltpu.async_copy(table_hbm.at[idx_v.at[s]], rows_v.at[slot],
                         sems.at[slot])

    def wait(slot, s):
        pltpu.make_async_copy(table_hbm.at[idx_v.at[s]], rows_v.at[slot],
                              sems.at[slot]).wait()

    def reduce_slot(slot, accs):
        def row_body(r, accs):
            return tuple(
                a + rows_v[slot, r, pl.ds(e * _LANES, _LANES)]
                for e, a in enumerate(accs)
            )
        return lax.fori_loop(0, _L, row_body, accs, unroll=8)

    for j in range(_NBUF):
        start(j, j)

    ngrp = _SEQ_PER_W // _NBUF

    def grp_body(g, carry):
        for j in range(_NBUF):
            s = _NBUF * g + j
            wait(j, s)

            @pl.when(g < ngrp - 1)
            def _():
                start(j, s + _NBUF)

            accs = tuple(jnp.zeros((_LANES,), jnp.float32)
                         for _ in range(nvec))
            accs = reduce_slot(j, accs)
            for e in range(nvec):
                outbuf[s, pl.ds(e * _LANES, _LANES)] = accs[e]
        return carry

    lax.fori_loop(0, ngrp, grp_body, 0)
    pltpu.sync_copy(outbuf, out_hbm.at[pl.ds(base, _SEQ_PER_W)])


_sc_gather_sum = functools.partial(
    pl.kernel,
    out_type=jax.ShapeDtypeStruct((_B, _EMBED), jnp.float32),
    mesh=plsc.VectorSubcoreMesh(
        core_axis_name="c", subcore_axis_name="s",
        num_cores=_NC, num_subcores=_NS),
    scratch_types=[
        pltpu.VMEM((_SEQ_PER_W, _L), jnp.int32),
        pltpu.VMEM((_NBUF, _L, _EMBED), jnp.float32),
        pltpu.VMEM((_SEQ_PER_W, _EMBED), jnp.float32),
        pltpu.SemaphoreType.DMA((_NBUF,)),
    ],
    compiler_params=pltpu.CompilerParams(use_tc_tiling_on_sc=False),
)(_sc_body)


_TBLK = 16384
_KSPLIT = 31 * _TBLK


def _tr_body(x0_ref, x1_ref, out_ref):
    out_ref[:, 0:_EMBED] = x0_ref[...].T
    out_ref[:, _EMBED:2 * _EMBED] = x1_ref[...].T


def _tc_transpose(emb_t):
    grid = _KSPLIT // _TBLK
    return pl.pallas_call(
        _tr_body,
        grid=(grid,),
        in_specs=[
            pl.BlockSpec((_EMBED, _TBLK), lambda i: (0, i)),
            pl.BlockSpec(
                (_EMBED, _TBLK),
                lambda i: (0, jnp.minimum(grid + i, (_VOCAB - 1) // _TBLK)),
            ),
        ],
        out_specs=pl.BlockSpec((_TBLK, 2 * _EMBED), lambda i: (i, 0)),
        out_shape=jax.ShapeDtypeStruct((_KSPLIT, 2 * _EMBED), jnp.float32),
    )(emb_t, emb_t)


def _tc_body(summed_ref, seq_ref, wt_ref, b_ref, out_ref):
    cnt = jnp.sum((seq_ref[...] != 0).astype(jnp.float32), axis=1,
                  keepdims=True)
    cnt = jnp.maximum(cnt, 1.0)
    acc = jnp.dot(summed_ref[...], wt_ref[...],
                  preferred_element_type=jnp.float32)
    out_ref[...] = acc / cnt + b_ref[...]


def _tc_classifier(summed, seqs, wt, b2):
    blk = 1024
    grid = _B // blk
    return pl.pallas_call(
        _tc_body,
        grid=(grid,),
        in_specs=[
            pl.BlockSpec((blk, _EMBED), lambda i: (i, 0)),
            pl.BlockSpec((blk, _L), lambda i: (i, 0)),
            pl.BlockSpec((_EMBED, _NCLS), lambda i: (0, 0)),
            pl.BlockSpec((1, _NCLS), lambda i: (0, 0)),
        ],
        out_specs=pl.BlockSpec((blk, _NCLS), lambda i: (i, 0)),
        out_shape=jax.ShapeDtypeStruct((_B, _NCLS), jnp.float32),
    )(summed, seqs, wt, b2)


def kernel(sequences, emb_table, W, b):
    seqs = sequences.astype(jnp.int32)
    wide = _tc_transpose(emb_table.T)
    t_lin = wide.reshape(2 * _KSPLIT, _EMBED)
    hi = (seqs >= _KSPLIT).astype(jnp.int32)
    gidx = 2 * (seqs - _KSPLIT * hi) + hi
    summed = _sc_gather_sum(t_lin, gidx)
    return _tc_classifier(summed, seqs, W.T, b.reshape(1, _NCLS))


# speedup vs baseline: 1.7767x; 1.0030x over previous
import functools

import jax
import jax.numpy as jnp
from jax import lax
from jax.experimental import pallas as pl
from jax.experimental.pallas import tpu as pltpu
from jax.experimental.pallas import tpu_sc as plsc

_VOCAB = 1000000
_EMBED = 64
_NCLS = 10
_B = 4096
_L = 200
_NC = 2
_NS = 16
_NW = _NC * _NS
_SEQ_PER_W = _B // _NW
_LANES = 16


_NBUF = 4


def _sc_body(table_hbm, idx_hbm, out_hbm, idx_v, rows_v, outbuf, sems):
    wid = lax.axis_index("s") * _NC + lax.axis_index("c")
    base = wid * _SEQ_PER_W
    pltpu.sync_copy(idx_hbm.at[pl.ds(base, _SEQ_PER_W)], idx_v)

    nvec = _EMBED // _LANES

    def start(slot, s):
        pltpu.async_copy(table_hbm.at[idx_v.at[s]], rows_v.at[slot],
                         sems.at[slot])

    def wait(slot, s):
        pltpu.make_async_copy(table_hbm.at[idx_v.at[s]], rows_v.at[slot],
                              sems.at[slot]).wait()

    def reduce_slot(slot, accs):
        def row_body(r, accs):
            return tuple(
                a + rows_v[slot, r, pl.ds(e * _LANES, _LANES)]
                for e, a in enumerate(accs)
            )
        return lax.fori_loop(0, _L, row_body, accs, unroll=8)

    for j in range(_NBUF):
        start(j, j)

    ngrp = _SEQ_PER_W // _NBUF

    def grp_body(g, carry):
        for j in range(_NBUF):
            s = _NBUF * g + j
            wait(j, s)

            @pl.when(g < ngrp - 1)
            def _():
                start(j, s + _NBUF)

            accs = tuple(jnp.zeros((_LANES,), jnp.float32)
                         for _ in range(nvec))
            accs = reduce_slot(j, accs)
            for e in range(nvec):
                outbuf[s, pl.ds(e * _LANES, _LANES)] = accs[e]
        return carry

    lax.fori_loop(0, ngrp, grp_body, 0)
    pltpu.sync_copy(outbuf, out_hbm.at[pl.ds(base, _SEQ_PER_W)])


_sc_gather_sum = functools.partial(
    pl.kernel,
    out_type=jax.ShapeDtypeStruct((_B, _EMBED), jnp.float32),
    mesh=plsc.VectorSubcoreMesh(
        core_axis_name="c", subcore_axis_name="s",
        num_cores=_NC, num_subcores=_NS),
    scratch_types=[
        pltpu.VMEM((_SEQ_PER_W, _L), jnp.int32),
        pltpu.VMEM((_NBUF, _L, _EMBED), jnp.float32),
        pltpu.VMEM((_SEQ_PER_W, _EMBED), jnp.float32),
        pltpu.SemaphoreType.DMA((_NBUF,)),
    ],
    compiler_params=pltpu.CompilerParams(use_tc_tiling_on_sc=False),
)(_sc_body)


_TBLK = 16384
_KSPLIT = 31 * _TBLK


def _tr_body(x0_ref, x1_ref, out_ref):
    out_ref[:, 0:_EMBED] = x0_ref[...].T
    out_ref[:, _EMBED:2 * _EMBED] = x1_ref[...].T


def _tc_transpose(emb_t):
    grid = _KSPLIT // _TBLK
    return pl.pallas_call(
        _tr_body,
        grid=(grid,),
        in_specs=[
            pl.BlockSpec((_EMBED, _TBLK), lambda i: (0, i)),
            pl.BlockSpec(
                (_EMBED, _TBLK),
                lambda i: (0, jnp.minimum(grid + i, (_VOCAB - 1) // _TBLK)),
            ),
        ],
        out_specs=pl.BlockSpec((_TBLK, 2 * _EMBED), lambda i: (i, 0)),
        out_shape=jax.ShapeDtypeStruct((_KSPLIT, 2 * _EMBED), jnp.float32),
    )(emb_t, emb_t)


def _tc_body(summed_ref, seq_ref, wt_ref, b_ref, out_ref):
    cnt = jnp.sum((seq_ref[...] != 0).astype(jnp.float32), axis=1,
                  keepdims=True)
    cnt = jnp.maximum(cnt, 1.0)
    acc = jnp.dot(summed_ref[...], wt_ref[...],
                  preferred_element_type=jnp.float32)
    out_ref[...] = acc / cnt + b_ref[...]


def _tc_classifier(summed, seqs, wt, b2):
    blk = 1024
    grid = _B // blk
    return pl.pallas_call(
        _tc_body,
        grid=(grid,),
        in_specs=[
            pl.BlockSpec((blk, _EMBED), lambda i: (i, 0)),
            pl.BlockSpec((blk, _L), lambda i: (i, 0)),
            pl.BlockSpec((_EMBED, _NCLS), lambda i: (0, 0)),
            pl.BlockSpec((1, _NCLS), lambda i: (0, 0)),
        ],
        out_specs=pl.BlockSpec((blk, _NCLS), lambda i: (i, 0)),
        out_shape=jax.ShapeDtypeStruct((_B, _NCLS), jnp.float32),
    )(summed, seqs, wt, b2)


def kernel(sequences, emb_table, W, b):
    seqs = sequences.astype(jnp.int32)
    wide = _tc_transpose(emb_table.T)
    t_lin = wide.reshape(2 * _KSPLIT, _EMBED)
    hi = (seqs >= _KSPLIT).astype(jnp.int32)
    gidx = 2 * (seqs - _KSPLIT * hi) + hi
    summed = _sc_gather_sum(t_lin, gidx)
    return _tc_classifier(summed, seqs, W.T, b.reshape(1, _NCLS))
